# baseline (speedup 1.0000x reference)
import jax
import jax.numpy as jnp
from jax.experimental import pallas as pl
from jax.experimental.pallas import tpu as pltpu

_BF = 512


def _moe_body(x_ref, wg_ref, bg_ref, we_ref, be_ref, out_ref, g_scr):
    f = pl.program_id(0)
    e = pl.program_id(1)
    n_exp = g_scr.shape[1]

    @pl.when((f == 0) & (e == 0))
    def _prep():
        logits = jnp.dot(x_ref[...], wg_ref[...],
                         preferred_element_type=jnp.float32)
        logits = logits + bg_ref[...]
        m = jnp.max(logits, axis=-1, keepdims=True)
        p = jnp.exp(logits - m)
        g_scr[...] = p / jnp.sum(p, axis=-1, keepdims=True)

    lane = jax.lax.broadcasted_iota(jnp.int32, (1, n_exp), 1)
    ge = jnp.sum(jnp.where(lane == e, g_scr[...], 0.0), axis=-1, keepdims=True)

    mm = jnp.dot(x_ref[...], we_ref[0].astype(jnp.bfloat16),
                 preferred_element_type=jnp.float32)
    contrib = ge * (mm + be_ref[0])

    @pl.when(e == 0)
    def _init():
        out_ref[...] = contrib

    @pl.when(e != 0)
    def _acc():
        out_ref[...] += contrib


def kernel(x, Wg, bg, We, be):
    n, k = x.shape
    n_exp = Wg.shape[1]
    f_out = We.shape[2]
    bf = min(_BF, f_out)
    grid = (f_out // bf, n_exp)
    xb = x.astype(jnp.bfloat16)
    wgb = Wg.astype(jnp.bfloat16)
    return pl.pallas_call(
        _moe_body,
        grid=grid,
        in_specs=[
            pl.BlockSpec((n, k), lambda f, e: (0, 0)),
            pl.BlockSpec((k, n_exp), lambda f, e: (0, 0)),
            pl.BlockSpec((1, n_exp), lambda f, e: (0, 0)),
            pl.BlockSpec((1, k, bf), lambda f, e: (e, 0, f)),
            pl.BlockSpec((1, 1, bf), lambda f, e: (e, 0, f)),
        ],
        out_specs=pl.BlockSpec((n, bf), lambda f, e: (0, f)),
        out_shape=jax.ShapeDtypeStruct((n, f_out), jnp.float32),
        scratch_shapes=[
            pltpu.VMEM((n, n_exp), jnp.float32),
        ],
        compiler_params=pltpu.CompilerParams(
            dimension_semantics=("parallel", "arbitrary"),
        ),
    )(xb, wgb, bg.reshape(1, n_exp), We, be.reshape(n_exp, 1, f_out))
